# Initial kernel scaffold; baseline (speedup 1.0000x reference)
import functools

import jax
import jax.numpy as jnp
from jax import lax
from jax.experimental import pallas as pl
from jax.experimental.pallas import tpu as pltpu
from jax.experimental.pallas import tpu_sc as plsc

_NC = 2
_NS = 16
_EB = 80


def _sc_mesh():
    return plsc.VectorSubcoreMesh(core_axis_name="c", subcore_axis_name="s",
                                  num_cores=_NC, num_subcores=_NS)


def _deg_hist(edge_dst, n_nodes):
    e = edge_dst.shape[0]
    eps = e // (_NC * _NS)

    @functools.partial(
        pl.kernel,
        out_type=jax.ShapeDtypeStruct((_NC * _NS, n_nodes), jnp.float32),
        mesh=_sc_mesh(),
        scratch_types=[
            pltpu.VMEM((n_nodes,), jnp.float32),
            pltpu.VMEM((eps,), jnp.int32),
        ],
    )
    def k(dst_hbm, out_hbm, hist, didx):
        c = lax.axis_index("c")
        s = lax.axis_index("s")
        w = c * _NS + s

        @pl.loop(0, n_nodes // 16)
        def _zero(i):
            hist[pl.ds(i * 16, 16)] = jnp.zeros((16,), jnp.float32)

        pltpu.sync_copy(dst_hbm.at[pl.ds(w * eps, eps)], didx)
        ones = jnp.ones((16,), jnp.float32)

        @pl.loop(0, eps // 16)
        def _acc(i):
            idx = didx[pl.ds(i * 16, 16)]
            plsc.addupdate_scatter(hist, [idx], ones)

        pltpu.sync_copy(hist, out_hbm.at[w])

    return k(edge_dst)


def _gcn_aggregate(h_split, edge_index):
    n = h_split.shape[1]
    f = h_split.shape[2]
    e = edge_index.shape[1]
    eps = e // _NS
    nb = eps // _EB
    rps = n // _NS

    @functools.partial(
        pl.kernel,
        out_type=jax.ShapeDtypeStruct((_NC, n, f), jnp.float32),
        mesh=_sc_mesh(),
        scratch_types=[
            pltpu.VMEM_SHARED((n, f), jnp.float32),
            pltpu.VMEM((_EB,), jnp.int32),
            pltpu.VMEM((_EB,), jnp.int32),
            pltpu.VMEM((_EB, f), jnp.float32),
            pltpu.SemaphoreType.DMA,
        ],
    )
    def k(h_hbm, edge_hbm, out_hbm, acc, sidx, didx, rows, sem):
        c = lax.axis_index("c")
        s = lax.axis_index("s")
        r0 = s * rps
        pltpu.sync_copy(h_hbm.at[c, pl.ds(r0, rps), :],
                        acc.at[pl.ds(r0, rps), :])
        plsc.subcore_barrier()

        e0 = s * eps

        @pl.loop(0, nb)
        def _edges(i):
            off = e0 + i * _EB
            pltpu.sync_copy(edge_hbm.at[0, pl.ds(off, _EB)], sidx)
            pltpu.sync_copy(edge_hbm.at[1, pl.ds(off, _EB)], didx)
            pltpu.async_copy(h_hbm.at[c].at[sidx], rows, sem).wait()
            pltpu.sync_copy(rows, acc.at[didx], add=True)

        plsc.subcore_barrier()
        pltpu.sync_copy(acc.at[pl.ds(r0, rps), :],
                        out_hbm.at[c, pl.ds(r0, rps), :])

    return k(h_split, edge_index)


def _mm1(hists, x, w1):
    n = x.shape[0]
    hid = w1.shape[0]
    hf = hid // 2

    def body(h_ref, x_ref, w_ref, hp_ref, dinv_ref):
        deg = jnp.sum(h_ref[...], axis=0) + 1.0
        dinv = lax.rsqrt(deg)
        h = lax.dot_general(x_ref[...], w_ref[...],
                            (((1,), (1,)), ((), ())),
                            preferred_element_type=jnp.float32)
        hp = h * dinv[:, None]
        hp_ref[0] = hp[:, :hf]
        hp_ref[1] = hp[:, hf:]
        dinv_ref[...] = dinv

    return pl.pallas_call(
        body,
        out_shape=(jax.ShapeDtypeStruct((2, n, hf), jnp.float32),
                   jax.ShapeDtypeStruct((n,), jnp.float32)),
    )(hists, x, w1)


def _bn_relu(agg, dinv, b, g, be):
    n = agg.shape[1]
    hf = agg.shape[2]

    def body(a_ref, dinv_ref, b_ref, g_ref, be_ref, o_ref):
        dinv = dinv_ref[...]
        t = jnp.concatenate([a_ref[0], a_ref[1]], axis=1)
        t = t * dinv[:, None] + b_ref[...][None, :]
        mu = jnp.mean(t, axis=0)
        d = t - mu[None, :]
        var = jnp.mean(d * d, axis=0)
        y = g_ref[...][None, :] * d * lax.rsqrt(var + 1e-5) + be_ref[...][None, :]
        o_ref[...] = jnp.maximum(y, 0.0)

    return pl.pallas_call(
        body,
        out_shape=jax.ShapeDtypeStruct((n, 2 * hf), jnp.float32),
    )(agg, dinv, b, g, be)


def _mm2(y, w2, dinv):
    n = y.shape[0]
    out = w2.shape[0]
    hf = out // 2

    def body(y_ref, w_ref, dinv_ref, hp_ref):
        h = lax.dot_general(y_ref[...], w_ref[...],
                            (((1,), (1,)), ((), ())),
                            preferred_element_type=jnp.float32)
        hp = h * dinv_ref[...][:, None]
        hp_ref[0] = hp[:, :hf]
        hp_ref[1] = hp[:, hf:]

    return pl.pallas_call(
        body,
        out_shape=jax.ShapeDtypeStruct((2, n, hf), jnp.float32),
    )(y, w2, dinv)


def kernel(x, edge_index, W1, b1, g1, be1, W2, b2, g2, be2):
    edge_index = edge_index.astype(jnp.int32)
    n = x.shape[0]

    hists = _deg_hist(edge_index[1], n)
    h1p, dinv = _mm1(hists, x, W1)
    agg1 = _gcn_aggregate(h1p, edge_index)
    y1 = _bn_relu(agg1, dinv, b1, g1, be1)
    h2p = _mm2(y1, W2, dinv)
    agg2 = _gcn_aggregate(h2p, edge_index)
    return _bn_relu(agg2, dinv, b2, g2, be2)


# speedup vs baseline: 9.4407x; 9.4407x over previous
import functools

import jax
import jax.numpy as jnp
from jax import lax
from jax.experimental import pallas as pl
from jax.experimental.pallas import tpu as pltpu
from jax.experimental.pallas import tpu_sc as plsc

_NC = 2
_NS = 16
_EB = 80
_NP = 10240


def _sc_mesh():
    return plsc.VectorSubcoreMesh(core_axis_name="c", subcore_axis_name="s",
                                  num_cores=_NC, num_subcores=_NS)


def _deg_hist(edge_dst, n_nodes):
    e = edge_dst.shape[0]
    eps = e // (_NC * _NS)

    @functools.partial(
        pl.kernel,
        out_type=jax.ShapeDtypeStruct((_NC * _NS, n_nodes), jnp.float32),
        mesh=_sc_mesh(),
        scratch_types=[
            pltpu.VMEM((n_nodes,), jnp.float32),
            pltpu.VMEM((eps,), jnp.int32),
        ],
        compiler_params=pltpu.CompilerParams(needs_layout_passes=False),
    )
    def k(dst_hbm, out_hbm, hist, didx):
        c = lax.axis_index("c")
        s = lax.axis_index("s")
        w = c * _NS + s

        @pl.loop(0, n_nodes // 16)
        def _zero(i):
            hist[pl.ds(i * 16, 16)] = jnp.zeros((16,), jnp.float32)

        pltpu.sync_copy(dst_hbm.at[pl.ds(w * eps, eps)], didx)
        ones = jnp.ones((16,), jnp.float32)

        @pl.loop(0, eps // 16)
        def _acc(i):
            idx = didx[pl.ds(i * 16, 16)]
            plsc.addupdate_scatter(hist, [idx], ones)

        pltpu.sync_copy(hist, out_hbm.at[w])

    return k(edge_dst)


def _gcn_aggregate(h_split, edge_src, edge_dst):
    n = h_split.shape[1]
    f = h_split.shape[2]
    e = edge_src.shape[0]
    eps = e // _NS
    nb = eps // _EB
    rps = n // _NS

    @functools.partial(
        pl.kernel,
        out_type=jax.ShapeDtypeStruct((_NC, n, f), jnp.float32),
        mesh=_sc_mesh(),
        scratch_types=[
            pltpu.VMEM_SHARED((n, f), jnp.float32),
            pltpu.VMEM((_EB,), jnp.int32),
            pltpu.VMEM((_EB,), jnp.int32),
            pltpu.VMEM((_EB, f), jnp.float32),
            pltpu.SemaphoreType.DMA,
        ],
        compiler_params=pltpu.CompilerParams(needs_layout_passes=False),
    )
    def k(h_hbm, src_hbm, dst_hbm, out_hbm, acc, sidx, didx, rows, sem):
        c = lax.axis_index("c")
        s = lax.axis_index("s")
        r0 = s * rps
        pltpu.sync_copy(h_hbm.at[c, pl.ds(r0, rps), :],
                        acc.at[pl.ds(r0, rps), :])
        plsc.subcore_barrier()

        e0 = s * eps

        @pl.loop(0, nb)
        def _edges(i):
            off = e0 + i * _EB
            pltpu.sync_copy(src_hbm.at[pl.ds(off, _EB)], sidx)
            pltpu.sync_copy(dst_hbm.at[pl.ds(off, _EB)], didx)
            pltpu.async_copy(h_hbm.at[c].at[sidx], rows, sem).wait()
            pltpu.sync_copy(rows, acc.at[didx], add=True)

        plsc.subcore_barrier()
        pltpu.sync_copy(acc.at[pl.ds(r0, rps), :],
                        out_hbm.at[c, pl.ds(r0, rps), :])

    return k(h_split, edge_src, edge_dst)


def _mm1(hists, x, w1):
    n = x.shape[0]
    hid = w1.shape[0]
    hf = hid // 2

    def body(h_ref, x_ref, w_ref, hp_ref, dinv_ref):
        deg = jnp.sum(h_ref[...], axis=0) + 1.0
        dinv = lax.rsqrt(deg)
        h = lax.dot_general(x_ref[...], w_ref[...],
                            (((1,), (1,)), ((), ())),
                            preferred_element_type=jnp.float32)
        hp = h * dinv[:, None]
        hp_ref[0, pl.ds(0, n), :] = hp[:, :hf]
        hp_ref[1, pl.ds(0, n), :] = hp[:, hf:]
        pad = jnp.zeros((_NP - n, hf), jnp.float32)
        hp_ref[0, pl.ds(n, _NP - n), :] = pad
        hp_ref[1, pl.ds(n, _NP - n), :] = pad
        dinv_ref[...] = dinv

    return pl.pallas_call(
        body,
        out_shape=(jax.ShapeDtypeStruct((2, _NP, hf), jnp.float32),
                   jax.ShapeDtypeStruct((n,), jnp.float32)),
    )(hists, x, w1)


def _bn_relu(agg, dinv, b, g, be):
    n = dinv.shape[0]
    hf = agg.shape[2]

    def body(a_ref, dinv_ref, b_ref, g_ref, be_ref, o_ref):
        dinv = dinv_ref[...]
        t = jnp.concatenate([a_ref[0, pl.ds(0, n), :],
                             a_ref[1, pl.ds(0, n), :]], axis=1)
        t = t * dinv[:, None] + b_ref[...][None, :]
        mu = jnp.mean(t, axis=0)
        d = t - mu[None, :]
        var = jnp.mean(d * d, axis=0)
        y = g_ref[...][None, :] * d * lax.rsqrt(var + 1e-5) + be_ref[...][None, :]
        o_ref[...] = jnp.maximum(y, 0.0)

    return pl.pallas_call(
        body,
        out_shape=jax.ShapeDtypeStruct((n, 2 * hf), jnp.float32),
    )(agg, dinv, b, g, be)


def _mm2(y, w2, dinv):
    n = y.shape[0]
    out = w2.shape[0]
    hf = out // 2

    def body(y_ref, w_ref, dinv_ref, hp_ref):
        h = lax.dot_general(y_ref[...], w_ref[...],
                            (((1,), (1,)), ((), ())),
                            preferred_element_type=jnp.float32)
        hp = h * dinv_ref[...][:, None]
        hp_ref[0, pl.ds(0, n), :] = hp[:, :hf]
        hp_ref[1, pl.ds(0, n), :] = hp[:, hf:]
        pad = jnp.zeros((_NP - n, hf), jnp.float32)
        hp_ref[0, pl.ds(n, _NP - n), :] = pad
        hp_ref[1, pl.ds(n, _NP - n), :] = pad

    return pl.pallas_call(
        body,
        out_shape=jax.ShapeDtypeStruct((2, _NP, hf), jnp.float32),
    )(y, w2, dinv)


def kernel(x, edge_index, W1, b1, g1, be1, W2, b2, g2, be2):
    edge_index = edge_index.astype(jnp.int32)
    edge_src = edge_index[0]
    edge_dst = edge_index[1]
    n = x.shape[0]

    hists = _deg_hist(edge_dst, n)
    h1p, dinv = _mm1(hists, x, W1)
    agg1 = _gcn_aggregate(h1p, edge_src, edge_dst)
    y1 = _bn_relu(agg1, dinv, b1, g1, be1)
    h2p = _mm2(y1, W2, dinv)
    agg2 = _gcn_aggregate(h2p, edge_src, edge_dst)
    return _bn_relu(agg2, dinv, b2, g2, be2)


# speedup vs baseline: 25.9215x; 2.7457x over previous
import functools

import jax
import jax.numpy as jnp
from jax import lax
from jax.experimental import pallas as pl
from jax.experimental.pallas import tpu as pltpu
from jax.experimental.pallas import tpu_sc as plsc

_NC = 2
_NS = 16
_EB = 100
_NB = 3
_CH = 40


def _sc_mesh():
    return plsc.VectorSubcoreMesh(core_axis_name="c", subcore_axis_name="s",
                                  num_cores=_NC, num_subcores=_NS)


def _deg_hist(edge_dst, n_nodes):
    e = edge_dst.shape[0]
    eps = e // (_NC * _NS)

    @functools.partial(
        pl.kernel,
        out_type=jax.ShapeDtypeStruct((_NC * _NS, n_nodes), jnp.float32),
        mesh=_sc_mesh(),
        scratch_types=[
            pltpu.VMEM((n_nodes,), jnp.float32),
            pltpu.VMEM((eps,), jnp.int32),
        ],
        compiler_params=pltpu.CompilerParams(needs_layout_passes=False),
    )
    def k(dst_hbm, out_hbm, hist, didx):
        c = lax.axis_index("c")
        s = lax.axis_index("s")
        w = c * _NS + s

        @pl.loop(0, n_nodes // 16)
        def _zero(i):
            hist[pl.ds(i * 16, 16)] = jnp.zeros((16,), jnp.float32)

        pltpu.sync_copy(dst_hbm.at[pl.ds(w * eps, eps)], didx)
        ones = jnp.ones((16,), jnp.float32)

        @pl.loop(0, eps // 16)
        def _acc(i):
            idx = didx[pl.ds(i * 16, 16)]
            plsc.addupdate_scatter(hist, [idx], ones)

        pltpu.sync_copy(hist, out_hbm.at[w])

    return k(edge_dst)


def _gcn_aggregate(h_split, edge_src, edge_dst):
    n = h_split.shape[1]
    f = h_split.shape[2]
    nbt = edge_src.shape[0]
    nbs = nbt // _NS
    nch = nbs // _CH
    rps = (n // _NS) // 8 * 8
    rem = n - rps * _NS

    @functools.partial(
        pl.kernel,
        out_type=jax.ShapeDtypeStruct((_NC, n, f), jnp.float32),
        mesh=_sc_mesh(),
        scratch_types=[
            pltpu.VMEM_SHARED((n, f), jnp.float32),
            pltpu.VMEM((_CH, _EB), jnp.int32),
            pltpu.VMEM((_CH, _EB), jnp.int32),
            pltpu.VMEM((_NB, _EB, f), jnp.float32),
            [pltpu.SemaphoreType.DMA] * _NB,
            [pltpu.SemaphoreType.DMA] * _NB,
        ],
        compiler_params=pltpu.CompilerParams(needs_layout_passes=False),
    )
    def k(h_hbm, src_hbm, dst_hbm, out_hbm, acc, sidx, didx, rows,
          gsems, ssems):
        c = lax.axis_index("c")
        s = lax.axis_index("s")
        r0 = s * rps
        pltpu.sync_copy(h_hbm.at[c, pl.ds(r0, rps), :],
                        acc.at[pl.ds(r0, rps), :])

        @pl.when(s == 0)
        def _init_tail():
            pltpu.sync_copy(h_hbm.at[c, pl.ds(n - rem, rem), :],
                            acc.at[pl.ds(n - rem, rem), :])

        plsc.subcore_barrier()

        table = h_hbm.at[c]

        @pl.loop(0, nch)
        def _chunk(ch):
            row0 = s * nbs + ch * _CH
            pltpu.sync_copy(src_hbm.at[pl.ds(row0, _CH), :], sidx)
            pltpu.sync_copy(dst_hbm.at[pl.ds(row0, _CH), :], didx)
            gd, sd = {}, {}

            def scat(j):
                b = j % _NB
                gd[j].wait()
                sd[j] = pltpu.async_copy(rows.at[b], acc.at[didx.at[j]],
                                         ssems[b], add=True)

            for j in range(_CH):
                b = j % _NB
                if j >= _NB:
                    sd[j - _NB].wait()
                gd[j] = pltpu.async_copy(table.at[sidx.at[j]],
                                         rows.at[b], gsems[b])
                if j >= 1:
                    scat(j - 1)
            scat(_CH - 1)
            for j in range(_CH - _NB, _CH):
                sd[j].wait()

        plsc.subcore_barrier()
        pltpu.sync_copy(acc.at[pl.ds(r0, rps), :],
                        out_hbm.at[c, pl.ds(r0, rps), :])

        @pl.when(s == 0)
        def _out_tail():
            pltpu.sync_copy(acc.at[pl.ds(n - rem, rem), :],
                            out_hbm.at[c, pl.ds(n - rem, rem), :])

    return k(h_split, edge_src, edge_dst)


def _mm1(hists, x, w1):
    n = x.shape[0]
    hid = w1.shape[0]
    hf = hid // 2

    def body(h_ref, x_ref, w_ref, hp_ref, dinv_ref):
        deg = jnp.sum(h_ref[...], axis=0) + 1.0
        dinv = lax.rsqrt(deg)
        h = lax.dot_general(x_ref[...], w_ref[...],
                            (((1,), (1,)), ((), ())),
                            preferred_element_type=jnp.float32)
        hp = h * dinv[:, None]
        hp_ref[0] = hp[:, :hf]
        hp_ref[1] = hp[:, hf:]
        dinv_ref[...] = dinv

    return pl.pallas_call(
        body,
        out_shape=(jax.ShapeDtypeStruct((2, n, hf), jnp.float32),
                   jax.ShapeDtypeStruct((n,), jnp.float32)),
    )(hists, x, w1)


def _bn_relu(agg, dinv, b, g, be):
    n = dinv.shape[0]
    hf = agg.shape[2]

    def body(a_ref, dinv_ref, b_ref, g_ref, be_ref, o_ref):
        dinv = dinv_ref[...]
        t = jnp.concatenate([a_ref[0], a_ref[1]], axis=1)
        t = t * dinv[:, None] + b_ref[...][None, :]
        mu = jnp.mean(t, axis=0)
        d = t - mu[None, :]
        var = jnp.mean(d * d, axis=0)
        y = g_ref[...][None, :] * d * lax.rsqrt(var + 1e-5) + be_ref[...][None, :]
        o_ref[...] = jnp.maximum(y, 0.0)

    return pl.pallas_call(
        body,
        out_shape=jax.ShapeDtypeStruct((n, 2 * hf), jnp.float32),
    )(agg, dinv, b, g, be)


def _mm2(y, w2, dinv):
    n = y.shape[0]
    out = w2.shape[0]
    hf = out // 2

    def body(y_ref, w_ref, dinv_ref, hp_ref):
        h = lax.dot_general(y_ref[...], w_ref[...],
                            (((1,), (1,)), ((), ())),
                            preferred_element_type=jnp.float32)
        hp = h * dinv_ref[...][:, None]
        hp_ref[0] = hp[:, :hf]
        hp_ref[1] = hp[:, hf:]

    return pl.pallas_call(
        body,
        out_shape=jax.ShapeDtypeStruct((2, n, hf), jnp.float32),
    )(y, w2, dinv)


def kernel(x, edge_index, W1, b1, g1, be1, W2, b2, g2, be2):
    edge_index = edge_index.astype(jnp.int32)
    e = edge_index.shape[1]
    edge_src = edge_index[0].reshape(e // _EB, _EB)
    edge_dst = edge_index[1].reshape(e // _EB, _EB)
    n = x.shape[0]

    hists = _deg_hist(edge_index[1], n)
    h1p, dinv = _mm1(hists, x, W1)
    agg1 = _gcn_aggregate(h1p, edge_src, edge_dst)
    y1 = _bn_relu(agg1, dinv, b1, g1, be1)
    h2p = _mm2(y1, W2, dinv)
    agg2 = _gcn_aggregate(h2p, edge_src, edge_dst)
    return _bn_relu(agg2, dinv, b2, g2, be2)


# speedup vs baseline: 26.4983x; 1.0223x over previous
import functools

import jax
import jax.numpy as jnp
from jax import lax
from jax.experimental import pallas as pl
from jax.experimental.pallas import tpu as pltpu
from jax.experimental.pallas import tpu_sc as plsc

_NC = 2
_NS = 16
_EB = 100
_NB = 3
_CH = 40


def _sc_mesh():
    return plsc.VectorSubcoreMesh(core_axis_name="c", subcore_axis_name="s",
                                  num_cores=_NC, num_subcores=_NS)


def _deg_hist(edge_dst, n_nodes):
    e = edge_dst.shape[0]
    eps = e // (_NC * _NS)

    @functools.partial(
        pl.kernel,
        out_type=jax.ShapeDtypeStruct((_NC * _NS, n_nodes), jnp.float32),
        mesh=_sc_mesh(),
        scratch_types=[
            pltpu.VMEM((n_nodes,), jnp.float32),
            pltpu.VMEM((eps,), jnp.int32),
        ],
        compiler_params=pltpu.CompilerParams(needs_layout_passes=False),
    )
    def k(dst_hbm, out_hbm, hist, didx):
        c = lax.axis_index("c")
        s = lax.axis_index("s")
        w = c * _NS + s

        @pl.loop(0, n_nodes // 16)
        def _zero(i):
            hist[pl.ds(i * 16, 16)] = jnp.zeros((16,), jnp.float32)

        pltpu.sync_copy(dst_hbm.at[pl.ds(w * eps, eps)], didx)
        ones = jnp.ones((16,), jnp.float32)

        @pl.loop(0, eps // 16)
        def _acc(i):
            idx = didx[pl.ds(i * 16, 16)]
            plsc.addupdate_scatter(hist, [idx], ones)

        pltpu.sync_copy(hist, out_hbm.at[w])

    return k(edge_dst)


def _gcn_aggregate(h_split, edge_src, edge_dst):
    n = h_split.shape[1]
    f = h_split.shape[2]
    nbt = edge_src.shape[0]
    nbs = nbt // _NS
    nch = nbs // _CH
    rps = (n // _NS) // 8 * 8
    rem = n - rps * _NS

    @functools.partial(
        pl.kernel,
        out_type=jax.ShapeDtypeStruct((_NC, n, f), jnp.float32),
        mesh=_sc_mesh(),
        scratch_types=[
            pltpu.VMEM_SHARED((n, f), jnp.float32),
            pltpu.VMEM((_CH, _EB), jnp.int32),
            pltpu.VMEM((_CH, _EB), jnp.int32),
            pltpu.VMEM((_NB, _EB, f), jnp.float32),
            [pltpu.SemaphoreType.DMA] * _NB,
            [pltpu.SemaphoreType.DMA] * _NB,
        ],
        compiler_params=pltpu.CompilerParams(needs_layout_passes=False),
    )
    def k(h_hbm, src_hbm, dst_hbm, out_hbm, acc, sidx, didx, rows,
          gsems, ssems):
        c = lax.axis_index("c")
        s = lax.axis_index("s")
        r0 = s * rps
        pltpu.sync_copy(h_hbm.at[c, pl.ds(r0, rps), :],
                        acc.at[pl.ds(r0, rps), :])

        @pl.when(s == 0)
        def _init_tail():
            pltpu.sync_copy(h_hbm.at[c, pl.ds(n - rem, rem), :],
                            acc.at[pl.ds(n - rem, rem), :])

        plsc.subcore_barrier()

        table = h_hbm.at[c]

        @pl.loop(0, nch)
        def _chunk(ch):
            row0 = s * nbs + ch * _CH
            pltpu.sync_copy(src_hbm.at[pl.ds(row0, _CH), :], sidx)
            pltpu.sync_copy(dst_hbm.at[pl.ds(row0, _CH), :], didx)
            gd, sd = {}, {}

            def scat(j):
                b = j % _NB
                gd[j].wait()
                sd[j] = pltpu.async_copy(rows.at[b], acc.at[didx.at[j]],
                                         ssems[b], add=True)

            for j in range(_CH):
                b = j % _NB
                if j >= _NB:
                    sd[j - _NB].wait()
                gd[j] = pltpu.async_copy(table.at[sidx.at[j]],
                                         rows.at[b], gsems[b])
                if j >= 1:
                    scat(j - 1)
            scat(_CH - 1)
            for j in range(_CH - _NB, _CH):
                sd[j].wait()

        plsc.subcore_barrier()
        pltpu.sync_copy(acc.at[pl.ds(r0, rps), :],
                        out_hbm.at[c, pl.ds(r0, rps), :])

        @pl.when(s == 0)
        def _out_tail():
            pltpu.sync_copy(acc.at[pl.ds(n - rem, rem), :],
                            out_hbm.at[c, pl.ds(n - rem, rem), :])

    return k(h_split, edge_src, edge_dst)


def _mm1(hists, x, w1):
    n = x.shape[0]
    hid = w1.shape[0]
    hf = hid // 2

    def body(h_ref, x_ref, w_ref, hp_ref, dinv_ref):
        deg = jnp.sum(h_ref[...], axis=0) + 1.0
        dinv = lax.rsqrt(deg)
        h = lax.dot_general(x_ref[...], w_ref[...],
                            (((1,), (1,)), ((), ())),
                            preferred_element_type=jnp.float32)
        hp = h * dinv[:, None]
        hp_ref[0] = hp[:, :hf]
        hp_ref[1] = hp[:, hf:]
        dinv_ref[...] = dinv

    return pl.pallas_call(
        body,
        out_shape=(jax.ShapeDtypeStruct((2, n, hf), jnp.float32),
                   jax.ShapeDtypeStruct((n,), jnp.float32)),
    )(hists, x, w1)


def _bn_relu(agg, dinv, b, g, be):
    n = dinv.shape[0]
    hf = agg.shape[2]

    def body(a_ref, dinv_ref, b_ref, g_ref, be_ref, o_ref):
        dinv = dinv_ref[...]
        t = jnp.concatenate([a_ref[0], a_ref[1]], axis=1)
        t = t * dinv[:, None] + b_ref[...][None, :]
        mu = jnp.mean(t, axis=0)
        d = t - mu[None, :]
        var = jnp.mean(d * d, axis=0)
        y = g_ref[...][None, :] * d * lax.rsqrt(var + 1e-5) + be_ref[...][None, :]
        o_ref[...] = jnp.maximum(y, 0.0)

    return pl.pallas_call(
        body,
        out_shape=jax.ShapeDtypeStruct((n, 2 * hf), jnp.float32),
    )(agg, dinv, b, g, be)


def _bn_relu_mm2(agg, dinv, b, g, be, w2):
    n = agg.shape[1]
    out = w2.shape[0]
    hf = out // 2

    def body(a_ref, dinv_ref, b_ref, g_ref, be_ref, w_ref, hp_ref):
        dinv = dinv_ref[...]
        t = jnp.concatenate([a_ref[0], a_ref[1]], axis=1)
        t = t * dinv[:, None] + b_ref[...][None, :]
        mu = jnp.mean(t, axis=0)
        d = t - mu[None, :]
        var = jnp.mean(d * d, axis=0)
        y = g_ref[...][None, :] * d * lax.rsqrt(var + 1e-5) + be_ref[...][None, :]
        y = jnp.maximum(y, 0.0)
        h = lax.dot_general(y, w_ref[...],
                            (((1,), (1,)), ((), ())),
                            preferred_element_type=jnp.float32)
        hp = h * dinv[:, None]
        hp_ref[0] = hp[:, :hf]
        hp_ref[1] = hp[:, hf:]

    return pl.pallas_call(
        body,
        out_shape=jax.ShapeDtypeStruct((2, n, hf), jnp.float32),
    )(agg, dinv, b, g, be, w2)


def kernel(x, edge_index, W1, b1, g1, be1, W2, b2, g2, be2):
    edge_index = edge_index.astype(jnp.int32)
    e = edge_index.shape[1]
    edge_src = edge_index[0].reshape(e // _EB, _EB)
    edge_dst = edge_index[1].reshape(e // _EB, _EB)
    n = x.shape[0]

    hists = _deg_hist(edge_index[1], n)
    h1p, dinv = _mm1(hists, x, W1)
    agg1 = _gcn_aggregate(h1p, edge_src, edge_dst)
    h2p = _bn_relu_mm2(agg1, dinv, b1, g1, be1, W2)
    agg2 = _gcn_aggregate(h2p, edge_src, edge_dst)
    return _bn_relu(agg2, dinv, b2, g2, be2)


# speedup vs baseline: 27.2551x; 1.0286x over previous
import functools

import jax
import jax.numpy as jnp
from jax import lax
from jax.experimental import pallas as pl
from jax.experimental.pallas import tpu as pltpu
from jax.experimental.pallas import tpu_sc as plsc

_NC = 2
_NS = 16
_EB = 100
_NB = 3
_CH = 40


def _sc_mesh():
    return plsc.VectorSubcoreMesh(core_axis_name="c", subcore_axis_name="s",
                                  num_cores=_NC, num_subcores=_NS)


def _deg_hist(edge_dst, n_nodes):
    e = edge_dst.shape[0]
    eps = e // (_NC * _NS)

    @functools.partial(
        pl.kernel,
        out_type=jax.ShapeDtypeStruct((_NC * _NS, n_nodes), jnp.float32),
        mesh=_sc_mesh(),
        scratch_types=[
            pltpu.VMEM((n_nodes,), jnp.float32),
            pltpu.VMEM((eps,), jnp.int32),
        ],
        compiler_params=pltpu.CompilerParams(needs_layout_passes=False),
    )
    def k(dst_hbm, out_hbm, hist, didx):
        c = lax.axis_index("c")
        s = lax.axis_index("s")
        w = c * _NS + s

        @pl.loop(0, n_nodes // 16)
        def _zero(i):
            hist[pl.ds(i * 16, 16)] = jnp.zeros((16,), jnp.float32)

        pltpu.sync_copy(dst_hbm.at[pl.ds(w * eps, eps)], didx)
        ones = jnp.ones((16,), jnp.float32)

        @pl.loop(0, eps // 16)
        def _acc(i):
            idx = didx[pl.ds(i * 16, 16)]
            plsc.addupdate_scatter(hist, [idx], ones)

        pltpu.sync_copy(hist, out_hbm.at[w])

    return k(edge_dst)


def _gcn_aggregate(h_split, edge_src, edge_dst):
    n = h_split.shape[1]
    f = h_split.shape[2]
    nbt = edge_src.shape[0]
    nbs = nbt // _NS
    nch = nbs // _CH
    rps = (n // _NS) // 8 * 8
    rem = n - rps * _NS

    @functools.partial(
        pl.kernel,
        out_type=jax.ShapeDtypeStruct((_NC, n, f), jnp.float32),
        mesh=_sc_mesh(),
        scratch_types=[
            pltpu.VMEM_SHARED((n, f), jnp.float32),
            pltpu.VMEM((_CH, _EB), jnp.int32),
            pltpu.VMEM((_CH, _EB), jnp.int32),
            pltpu.VMEM((_NB, _EB, f), jnp.float32),
            [pltpu.SemaphoreType.DMA] * _NB,
            [pltpu.SemaphoreType.DMA] * _NB,
        ],
        compiler_params=pltpu.CompilerParams(needs_layout_passes=False),
    )
    def k(h_hbm, src_hbm, dst_hbm, out_hbm, acc, sidx, didx, rows,
          gsems, ssems):
        c = lax.axis_index("c")
        s = lax.axis_index("s")
        r0 = s * rps
        pltpu.sync_copy(h_hbm.at[c, pl.ds(r0, rps), :],
                        acc.at[pl.ds(r0, rps), :])

        @pl.when(s == 0)
        def _init_tail():
            pltpu.sync_copy(h_hbm.at[c, pl.ds(n - rem, rem), :],
                            acc.at[pl.ds(n - rem, rem), :])

        plsc.subcore_barrier()

        table = h_hbm.at[c]

        @pl.loop(0, nch)
        def _chunk(ch):
            row0 = s * nbs + ch * _CH
            pltpu.sync_copy(src_hbm.at[pl.ds(row0, _CH), :], sidx)
            pltpu.sync_copy(dst_hbm.at[pl.ds(row0, _CH), :], didx)
            gd, sd = {}, {}

            def scat(j):
                b = j % _NB
                gd[j].wait()
                sd[j] = pltpu.async_copy(rows.at[b], acc.at[didx.at[j]],
                                         ssems[b], add=True)

            for j in range(_CH):
                b = j % _NB
                if j >= _NB:
                    sd[j - _NB].wait()
                gd[j] = pltpu.async_copy(table.at[sidx.at[j]],
                                         rows.at[b], gsems[b])
                if j >= _NB - 1:
                    scat(j - (_NB - 1))
            for j in range(_CH - _NB + 1, _CH):
                scat(j)
            for j in range(_CH - _NB, _CH):
                sd[j].wait()

        plsc.subcore_barrier()
        pltpu.sync_copy(acc.at[pl.ds(r0, rps), :],
                        out_hbm.at[c, pl.ds(r0, rps), :])

        @pl.when(s == 0)
        def _out_tail():
            pltpu.sync_copy(acc.at[pl.ds(n - rem, rem), :],
                            out_hbm.at[c, pl.ds(n - rem, rem), :])

    return k(h_split, edge_src, edge_dst)


def _mm1(hists, x, w1):
    n = x.shape[0]
    hid = w1.shape[0]
    hf = hid // 2

    def body(h_ref, x_ref, w_ref, hp_ref, dinv_ref):
        deg = jnp.sum(h_ref[...], axis=0) + 1.0
        dinv = lax.rsqrt(deg)
        h = lax.dot_general(x_ref[...], w_ref[...],
                            (((1,), (1,)), ((), ())),
                            preferred_element_type=jnp.float32)
        hp = h * dinv[:, None]
        hp_ref[0] = hp[:, :hf]
        hp_ref[1] = hp[:, hf:]
        dinv_ref[...] = dinv

    return pl.pallas_call(
        body,
        out_shape=(jax.ShapeDtypeStruct((2, n, hf), jnp.float32),
                   jax.ShapeDtypeStruct((n,), jnp.float32)),
    )(hists, x, w1)


def _bn_relu(agg, dinv, b, g, be):
    n = dinv.shape[0]
    hf = agg.shape[2]

    def body(a_ref, dinv_ref, b_ref, g_ref, be_ref, o_ref):
        dinv = dinv_ref[...]
        t = jnp.concatenate([a_ref[0], a_ref[1]], axis=1)
        t = t * dinv[:, None] + b_ref[...][None, :]
        mu = jnp.mean(t, axis=0)
        d = t - mu[None, :]
        var = jnp.mean(d * d, axis=0)
        y = g_ref[...][None, :] * d * lax.rsqrt(var + 1e-5) + be_ref[...][None, :]
        o_ref[...] = jnp.maximum(y, 0.0)

    return pl.pallas_call(
        body,
        out_shape=jax.ShapeDtypeStruct((n, 2 * hf), jnp.float32),
    )(agg, dinv, b, g, be)


def _bn_relu_mm2(agg, dinv, b, g, be, w2):
    n = agg.shape[1]
    out = w2.shape[0]
    hf = out // 2

    def body(a_ref, dinv_ref, b_ref, g_ref, be_ref, w_ref, hp_ref):
        dinv = dinv_ref[...]
        t = jnp.concatenate([a_ref[0], a_ref[1]], axis=1)
        t = t * dinv[:, None] + b_ref[...][None, :]
        mu = jnp.mean(t, axis=0)
        d = t - mu[None, :]
        var = jnp.mean(d * d, axis=0)
        y = g_ref[...][None, :] * d * lax.rsqrt(var + 1e-5) + be_ref[...][None, :]
        y = jnp.maximum(y, 0.0)
        h = lax.dot_general(y, w_ref[...],
                            (((1,), (1,)), ((), ())),
                            preferred_element_type=jnp.float32)
        hp = h * dinv[:, None]
        hp_ref[0] = hp[:, :hf]
        hp_ref[1] = hp[:, hf:]

    return pl.pallas_call(
        body,
        out_shape=jax.ShapeDtypeStruct((2, n, hf), jnp.float32),
    )(agg, dinv, b, g, be, w2)


def kernel(x, edge_index, W1, b1, g1, be1, W2, b2, g2, be2):
    edge_index = edge_index.astype(jnp.int32)
    e = edge_index.shape[1]
    edge_src = edge_index[0].reshape(e // _EB, _EB)
    edge_dst = edge_index[1].reshape(e // _EB, _EB)
    n = x.shape[0]

    hists = _deg_hist(edge_index[1], n)
    h1p, dinv = _mm1(hists, x, W1)
    agg1 = _gcn_aggregate(h1p, edge_src, edge_dst)
    h2p = _bn_relu_mm2(agg1, dinv, b1, g1, be1, W2)
    agg2 = _gcn_aggregate(h2p, edge_src, edge_dst)
    return _bn_relu(agg2, dinv, b2, g2, be2)
